# baseline (speedup 1.0000x reference)
import functools

import jax
import jax.numpy as jnp
from jax import lax
from jax.experimental import pallas as pl
from jax.experimental.pallas import tpu as pltpu
from jax.experimental.pallas import tpu_sc as plsc

_DIM = 32
_EPS = 1e-06
_MARGIN = 1.0
_GRP = 16
_CHUNK = 128


def _sc_body(bpw, nc, e4, r4, ph, pr, pt, nh, nr, nt, outp, outn,
             hs, rs, ts, hq, rq, tq, blkh, blkr, blkt, acc, sem):
    wid = lax.axis_index("s") * nc + lax.axis_index("c")
    base = wid * bpw
    sl = pl.ds(base, bpw)
    n_chunks = bpw // _CHUNK

    def side(h_hbm, r_hbm, t_hbm, out_hbm):
        pltpu.sync_copy(h_hbm.at[sl], hs)
        pltpu.sync_copy(r_hbm.at[sl], rs)
        pltpu.sync_copy(t_hbm.at[sl], ts)

        def qpass(g, carry):
            s = pl.ds(g * _GRP, _GRP)
            hq[s] = hs[s] >> 2
            rq[s] = rs[s] >> 2
            tq[s] = ts[s] >> 2
            return carry

        lax.fori_loop(0, bpw // _GRP, qpass, 0)

        def chunk(c, carry):
            cs = pl.ds(c * _CHUNK, _CHUNK)
            copies = [
                pltpu.async_copy(e4.at[hq.at[cs]], blkh, sem),
                pltpu.async_copy(r4.at[rq.at[cs]], blkr, sem),
                pltpu.async_copy(e4.at[tq.at[cs]], blkt, sem),
            ]
            for cp in copies:
                cp.wait()

            def group(g, carry2):
                gb = c * _CHUNK + g * _GRP
                hv = hs[pl.ds(gb, _GRP)]
                rv = rs[pl.ds(gb, _GRP)]
                tv = ts[pl.ds(gb, _GRP)]
                ho = (hv & 3) * 32
                ro = (rv & 3) * 32
                to = (tv & 3) * 32
                for kk in range(_GRP):
                    row = g * _GRP + kk
                    oh = ho[kk]
                    orr = ro[kk]
                    ot = to[kk]
                    h_lo = blkh[row, pl.ds(oh, 16)]
                    h_hi = blkh[row, pl.ds(oh + 16, 16)]
                    r_lo = blkr[row, pl.ds(orr, 16)]
                    r_hi = blkr[row, pl.ds(orr + 16, 16)]
                    t_lo = blkt[row, pl.ds(ot, 16)]
                    t_hi = blkt[row, pl.ds(ot + 16, 16)]
                    d_lo = h_lo + r_lo - t_lo + _EPS
                    d_hi = h_hi + r_hi - t_hi + _EPS
                    acc[gb + kk, :] = d_lo * d_lo + d_hi * d_hi
                return carry2

            lax.fori_loop(0, _CHUNK // _GRP, group, 0)
            return carry

        lax.fori_loop(0, n_chunks, chunk, 0)
        pltpu.sync_copy(acc, out_hbm.at[sl, :])

    side(ph, pr, pt, outp)
    side(nh, nr, nt, outn)


@functools.lru_cache(maxsize=None)
def _make_sc_kernel(batch):
    info = plsc.get_sparse_core_info()
    nc, ns = info.num_cores, info.num_subcores
    nw = nc * ns
    assert batch % (nw * _CHUNK) == 0
    bpw = batch // nw
    mesh = plsc.VectorSubcoreMesh(core_axis_name="c", subcore_axis_name="s")
    return pl.kernel(
        functools.partial(_sc_body, bpw, nc),
        out_type=[jax.ShapeDtypeStruct((batch, _GRP), jnp.float32),
                  jax.ShapeDtypeStruct((batch, _GRP), jnp.float32)],
        mesh=mesh,
        compiler_params=pltpu.CompilerParams(use_tc_tiling_on_sc=False),
        scratch_types=(
            [pltpu.VMEM((bpw,), jnp.int32)] * 6
            + [pltpu.VMEM((_CHUNK, 128), jnp.float32)] * 3
            + [pltpu.VMEM((bpw, _GRP), jnp.float32)]
            + [pltpu.SemaphoreType.DMA]
        ),
    )


def _tc_body(batch, p_ref, n_ref, out_ref):
    ps = jnp.sum(p_ref[...], axis=1)
    ns = jnp.sum(n_ref[...], axis=1)
    hinge = jnp.maximum(jnp.sqrt(ps) - jnp.sqrt(ns) + _MARGIN, 0.0)
    out_ref[0, 0] = jnp.sum(hinge) / batch


def kernel(pos_x, neg_x, ent_emb, rel_emb):
    batch = pos_x.shape[0]
    nrows, dim = ent_emb.shape
    e4 = ent_emb.reshape(nrows // 4, 4 * dim)
    r4 = rel_emb.reshape(nrows // 4, 4 * dim)
    ph, pr, pt = pos_x[:, 0], pos_x[:, 1], pos_x[:, 2]
    nh, nr, nt = neg_x[:, 0], neg_x[:, 1], neg_x[:, 2]
    pos_sq, neg_sq = _make_sc_kernel(batch)(
        e4, r4, ph, pr, pt, nh, nr, nt)
    out = pl.pallas_call(
        functools.partial(_tc_body, batch),
        out_shape=jax.ShapeDtypeStruct((1, 1), jnp.float32),
        out_specs=pl.BlockSpec(memory_space=pltpu.SMEM),
    )(pos_sq, neg_sq)
    return out[0, 0]
